# baseline (speedup 1.0000x reference)
import jax
import jax.numpy as jnp
from jax.experimental import pallas as pl
from jax.experimental.pallas import tpu as pltpu

_EPS = 0.4
_TOPK = 10
_BLK = 5000
_BIGI = 1 << 30


def _topk_kernel(in_ref, grad_ref, keys_ref, vals_ref, idx_ref,
                 rv_ref, ri_ref, sim_ref):
    b = pl.program_id(0)
    nblk = pl.num_programs(0)
    nq = in_ref.shape[0]

    @pl.when(b == 0)
    def _init():
        rv_ref[...] = jnp.full((nq, _TOPK), -jnp.inf, jnp.float32)
        ri_ref[...] = jnp.zeros((nq, _TOPK), jnp.int32)

    q = in_ref[...] + _EPS * jnp.sign(grad_ref[...])
    qn = q / jnp.clip(jnp.sqrt(jnp.sum(q * q, axis=-1, keepdims=True)), 1e-12)
    kt = keys_ref[...].T
    n = jnp.clip(jnp.sqrt(jnp.sum(kt * kt, axis=0)), 1e-12)
    knt = kt / n[None, :]
    sim_ref[...] = jax.lax.dot_general(
        qn, knt, (((1,), (0,)), ((), ())), preferred_element_type=jnp.float32
    )

    col = jax.lax.broadcasted_iota(jnp.int32, (nq, _BLK), 1)
    c10 = jax.lax.broadcasted_iota(jnp.int32, (nq, _TOPK), 1)
    base = b * _BLK

    def cond(carry):
        m, rv, ri = carry
        return jnp.any(m > rv[:, _TOPK - 1])

    def body(carry):
        m, rv, ri = carry
        s = sim_ref[...]
        a = jnp.min(jnp.where(s == m[:, None], col, _BIGI), axis=1)
        gi = a + base
        need = m > rv[:, _TOPK - 1]
        pos = jnp.sum(rv >= m[:, None], axis=1)[:, None]
        sv = jnp.concatenate([rv[:, :1], rv[:, :-1]], axis=1)
        si = jnp.concatenate([ri[:, :1], ri[:, :-1]], axis=1)
        nrv = jnp.where(c10 < pos, rv, jnp.where(c10 == pos, m[:, None], sv))
        nri = jnp.where(c10 < pos, ri, jnp.where(c10 == pos, gi[:, None], si))
        nrv = jnp.where(need[:, None], nrv, rv)
        nri = jnp.where(need[:, None], nri, ri)
        s = jnp.where(col == a[:, None], -jnp.inf, s)
        sim_ref[...] = s
        return jnp.max(s, axis=1), nrv, nri

    m0 = jnp.max(sim_ref[...], axis=1)
    _, rv_fin, ri_fin = jax.lax.while_loop(
        cond, body, (m0, rv_ref[...], ri_ref[...])
    )
    rv_ref[...] = rv_fin
    ri_ref[...] = ri_fin

    @pl.when(b == nblk - 1)
    def _out():
        vals_ref[...] = rv_ref[...]
        idx_ref[...] = ri_ref[...]


def kernel(in_data, data_grad, keys):
    nq = in_data.shape[0]
    nblk = keys.shape[0] // _BLK
    return pl.pallas_call(
        _topk_kernel,
        grid=(nblk,),
        in_specs=[
            pl.BlockSpec((nq, 32), lambda i: (0, 0)),
            pl.BlockSpec((nq, 32), lambda i: (0, 0)),
            pl.BlockSpec((_BLK, 32), lambda i: (i, 0)),
        ],
        out_specs=[
            pl.BlockSpec((nq, _TOPK), lambda i: (0, 0)),
            pl.BlockSpec((nq, _TOPK), lambda i: (0, 0)),
        ],
        out_shape=[
            jax.ShapeDtypeStruct((nq, _TOPK), jnp.float32),
            jax.ShapeDtypeStruct((nq, _TOPK), jnp.int32),
        ],
        scratch_shapes=[
            pltpu.VMEM((nq, _TOPK), jnp.float32),
            pltpu.VMEM((nq, _TOPK), jnp.int32),
            pltpu.VMEM((nq, _BLK), jnp.float32),
        ],
        compiler_params=pltpu.CompilerParams(dimension_semantics=("arbitrary",)),
    )(in_data, data_grad, keys)


# speedup vs baseline: 1.0309x; 1.0309x over previous
import jax
import jax.numpy as jnp
from jax.experimental import pallas as pl
from jax.experimental.pallas import tpu as pltpu

_EPS = 0.4
_TOPK = 10
_BLK = 8000
_BIGI = 1 << 30


def _topk_kernel(in_ref, grad_ref, keys_ref, vals_ref, idx_ref,
                 rv_ref, ri_ref, sim_ref):
    b = pl.program_id(0)
    nblk = pl.num_programs(0)
    nq = in_ref.shape[0]

    @pl.when(b == 0)
    def _init():
        rv_ref[...] = jnp.full((nq, _TOPK), -jnp.inf, jnp.float32)
        ri_ref[...] = jnp.zeros((nq, _TOPK), jnp.int32)

    q = in_ref[...] + _EPS * jnp.sign(grad_ref[...])
    qn = q / jnp.clip(jnp.sqrt(jnp.sum(q * q, axis=-1, keepdims=True)), 1e-12)
    kt = keys_ref[...].T
    n = jnp.clip(jnp.sqrt(jnp.sum(kt * kt, axis=0)), 1e-12)
    knt = kt / n[None, :]
    sim_ref[...] = jax.lax.dot_general(
        qn, knt, (((1,), (0,)), ((), ())), preferred_element_type=jnp.float32
    )

    col = jax.lax.broadcasted_iota(jnp.int32, (nq, _BLK), 1)
    c10 = jax.lax.broadcasted_iota(jnp.int32, (nq, _TOPK), 1)
    base = b * _BLK

    def cond(carry):
        m, rv, ri = carry
        return jnp.any(m > rv[:, _TOPK - 1])

    def body(carry):
        m, rv, ri = carry
        s = sim_ref[...]
        for _ in range(2):
            a = jnp.min(jnp.where(s == m[:, None], col, _BIGI), axis=1)
            gi = a + base
            need = m > rv[:, _TOPK - 1]
            pos = jnp.sum(rv >= m[:, None], axis=1)[:, None]
            sv = jnp.concatenate([rv[:, :1], rv[:, :-1]], axis=1)
            si = jnp.concatenate([ri[:, :1], ri[:, :-1]], axis=1)
            nrv = jnp.where(c10 < pos, rv, jnp.where(c10 == pos, m[:, None], sv))
            nri = jnp.where(c10 < pos, ri, jnp.where(c10 == pos, gi[:, None], si))
            rv = jnp.where(need[:, None], nrv, rv)
            ri = jnp.where(need[:, None], nri, ri)
            s = jnp.where(col == a[:, None], -jnp.inf, s)
            m = jnp.max(s, axis=1)
        sim_ref[...] = s
        return m, rv, ri

    m0 = jnp.max(sim_ref[...], axis=1)
    _, rv_fin, ri_fin = jax.lax.while_loop(
        cond, body, (m0, rv_ref[...], ri_ref[...])
    )
    rv_ref[...] = rv_fin
    ri_ref[...] = ri_fin

    @pl.when(b == nblk - 1)
    def _out():
        vals_ref[...] = rv_ref[...]
        idx_ref[...] = ri_ref[...]


def kernel(in_data, data_grad, keys):
    nq = in_data.shape[0]
    nblk = keys.shape[0] // _BLK
    return pl.pallas_call(
        _topk_kernel,
        grid=(nblk,),
        in_specs=[
            pl.BlockSpec((nq, 32), lambda i: (0, 0)),
            pl.BlockSpec((nq, 32), lambda i: (0, 0)),
            pl.BlockSpec((_BLK, 32), lambda i: (i, 0)),
        ],
        out_specs=[
            pl.BlockSpec((nq, _TOPK), lambda i: (0, 0)),
            pl.BlockSpec((nq, _TOPK), lambda i: (0, 0)),
        ],
        out_shape=[
            jax.ShapeDtypeStruct((nq, _TOPK), jnp.float32),
            jax.ShapeDtypeStruct((nq, _TOPK), jnp.int32),
        ],
        scratch_shapes=[
            pltpu.VMEM((nq, _TOPK), jnp.float32),
            pltpu.VMEM((nq, _TOPK), jnp.int32),
            pltpu.VMEM((nq, _BLK), jnp.float32),
        ],
        compiler_params=pltpu.CompilerParams(dimension_semantics=("arbitrary",)),
    )(in_data, data_grad, keys)


# speedup vs baseline: 1.0515x; 1.0200x over previous
import jax
import jax.numpy as jnp
from jax.experimental import pallas as pl
from jax.experimental.pallas import tpu as pltpu

_EPS = 0.4
_TOPK = 10
_BLK = 8000
_BIGI = 1 << 30


def _topk_kernel(in_ref, grad_ref, keys_ref, vals_ref, idx_ref,
                 rv_ref, ri_ref, sim_ref):
    b = pl.program_id(0)
    nblk = pl.num_programs(0)
    nq = in_ref.shape[0]

    @pl.when(b == 0)
    def _init():
        rv_ref[...] = jnp.full((nq, _TOPK), -jnp.inf, jnp.float32)
        ri_ref[...] = jnp.zeros((nq, _TOPK), jnp.int32)

    q = in_ref[...] + _EPS * jnp.sign(grad_ref[...])
    qn = q / jnp.clip(jnp.sqrt(jnp.sum(q * q, axis=-1, keepdims=True)), 1e-12)
    kt = keys_ref[...].T
    n = jnp.clip(jnp.sqrt(jnp.sum(kt * kt, axis=0)), 1e-12)
    knt = kt / n[None, :]
    sim_ref[...] = jax.lax.dot_general(
        qn, knt, (((1,), (0,)), ((), ())), preferred_element_type=jnp.float32
    )

    col = jax.lax.broadcasted_iota(jnp.int32, (nq, _BLK), 1)
    c10 = jax.lax.broadcasted_iota(jnp.int32, (nq, _TOPK), 1)
    base = b * _BLK

    def cond(carry):
        m, rv, ri = carry
        return jnp.any(m > rv[:, _TOPK - 1])

    def body(carry):
        m, rv, ri = carry
        s = sim_ref[...]
        a = jnp.min(jnp.where(s == m[:, None], col, _BIGI), axis=1)
        gi = a + base
        need = m > rv[:, _TOPK - 1]
        pos = jnp.sum(rv >= m[:, None], axis=1)[:, None]
        sv = jnp.concatenate([rv[:, :1], rv[:, :-1]], axis=1)
        si = jnp.concatenate([ri[:, :1], ri[:, :-1]], axis=1)
        nrv = jnp.where(c10 < pos, rv, jnp.where(c10 == pos, m[:, None], sv))
        nri = jnp.where(c10 < pos, ri, jnp.where(c10 == pos, gi[:, None], si))
        nrv = jnp.where(need[:, None], nrv, rv)
        nri = jnp.where(need[:, None], nri, ri)
        s = jnp.where(col == a[:, None], -jnp.inf, s)
        sim_ref[...] = s
        return jnp.max(s, axis=1), nrv, nri

    m0 = jnp.max(sim_ref[...], axis=1)
    _, rv_fin, ri_fin = jax.lax.while_loop(
        cond, body, (m0, rv_ref[...], ri_ref[...])
    )
    rv_ref[...] = rv_fin
    ri_ref[...] = ri_fin

    @pl.when(b == nblk - 1)
    def _out():
        vals_ref[...] = rv_ref[...]
        idx_ref[...] = ri_ref[...]


def kernel(in_data, data_grad, keys):
    nq = in_data.shape[0]
    nblk = keys.shape[0] // _BLK
    return pl.pallas_call(
        _topk_kernel,
        grid=(nblk,),
        in_specs=[
            pl.BlockSpec((nq, 32), lambda i: (0, 0)),
            pl.BlockSpec((nq, 32), lambda i: (0, 0)),
            pl.BlockSpec((_BLK, 32), lambda i: (i, 0)),
        ],
        out_specs=[
            pl.BlockSpec((nq, _TOPK), lambda i: (0, 0)),
            pl.BlockSpec((nq, _TOPK), lambda i: (0, 0)),
        ],
        out_shape=[
            jax.ShapeDtypeStruct((nq, _TOPK), jnp.float32),
            jax.ShapeDtypeStruct((nq, _TOPK), jnp.int32),
        ],
        scratch_shapes=[
            pltpu.VMEM((nq, _TOPK), jnp.float32),
            pltpu.VMEM((nq, _TOPK), jnp.int32),
            pltpu.VMEM((nq, _BLK), jnp.float32),
        ],
        compiler_params=pltpu.CompilerParams(dimension_semantics=("arbitrary",)),
    )(in_data, data_grad, keys)
